# baseline (speedup 1.0000x reference)
import functools

import jax
import jax.numpy as jnp
from jax import lax
from jax.experimental import pallas as pl
from jax.experimental.pallas import tpu as pltpu
from jax.experimental.pallas import tpu_sc as plsc

_INFO = plsc.get_sparse_core_info()
_NC = _INFO.num_cores
_NS = _INFO.num_subcores
_NW = _NC * _NS

_CHUNK = 8
_NBUF = 12


@functools.lru_cache(maxsize=None)
def _make_gather(batch: int, seq: int, d_model: int):
    num_rows = batch * seq
    assert num_rows % _NW == 0
    b_per_w = num_rows // _NW
    assert seq % b_per_w == 0
    w_per_row = seq // b_per_w
    chunk = _CHUNK
    nbuf = _NBUF
    assert b_per_w % chunk == 0
    n_chunks = b_per_w // chunk

    mesh = plsc.VectorSubcoreMesh(core_axis_name="c", subcore_axis_name="s")

    @functools.partial(
        pl.kernel,
        mesh=mesh,
        out_type=jax.ShapeDtypeStruct((batch, seq, d_model), jnp.float32),
        scratch_types=(
            [pltpu.VMEM((b_per_w,), jnp.int32)]
            + [pltpu.VMEM((chunk, d_model), jnp.float32)] * nbuf
            + [pltpu.SemaphoreType.DMA] * (2 * nbuf)
        ),
    )
    def gather_kernel(idx_hbm, table_hbm, out_hbm, idx_v, *rest):
        bufs = rest[:nbuf]
        gsems = rest[nbuf:2 * nbuf]
        ssems = rest[2 * nbuf:]

        wid = lax.axis_index("s") * _NC + lax.axis_index("c")
        row = wid // w_per_row
        col = (wid % w_per_row) * b_per_w
        pltpu.sync_copy(idx_hbm.at[row, pl.ds(col, b_per_w)], idx_v)

        def start_gather(c):
            return pltpu.async_copy(
                table_hbm.at[idx_v.at[pl.ds(c * chunk, chunk)]],
                bufs[c % nbuf],
                gsems[c % nbuf],
            )

        def start_store(c):
            return pltpu.async_copy(
                bufs[c % nbuf],
                out_hbm.at[row, pl.ds(col + c * chunk, chunk)],
                ssems[c % nbuf],
            )

        gathers = [None] * n_chunks
        stores = [None] * n_chunks
        for c in range(min(nbuf - 1, n_chunks)):
            gathers[c] = start_gather(c)
        for c in range(n_chunks):
            if c + nbuf - 1 < n_chunks:
                if c >= 1:
                    stores[c - 1].wait()
                gathers[c + nbuf - 1] = start_gather(c + nbuf - 1)
            gathers[c].wait()
            stores[c] = start_store(c)
        for c in range(max(0, n_chunks - nbuf), n_chunks):
            stores[c].wait()

    return gather_kernel


def kernel(input_ids, table):
    batch, seq = input_ids.shape
    vocab, d_model = table.shape
    return _make_gather(batch, seq, d_model)(
        input_ids.astype(jnp.int32), table)


# speedup vs baseline: 1.0219x; 1.0219x over previous
import functools

import jax
import jax.numpy as jnp
from jax import lax
from jax.experimental import pallas as pl
from jax.experimental.pallas import tpu as pltpu
from jax.experimental.pallas import tpu_sc as plsc

_INFO = plsc.get_sparse_core_info()
_NC = _INFO.num_cores
_NS = _INFO.num_subcores
_NW = _NC * _NS

_CHUNK = 16
_NBUF = 7


@functools.lru_cache(maxsize=None)
def _make_gather(batch: int, seq: int, d_model: int):
    num_rows = batch * seq
    assert num_rows % _NW == 0
    b_per_w = num_rows // _NW
    assert seq % b_per_w == 0
    w_per_row = seq // b_per_w
    chunk = _CHUNK
    nbuf = _NBUF
    assert b_per_w % chunk == 0
    n_chunks = b_per_w // chunk

    mesh = plsc.VectorSubcoreMesh(core_axis_name="c", subcore_axis_name="s")

    @functools.partial(
        pl.kernel,
        mesh=mesh,
        out_type=jax.ShapeDtypeStruct((batch, seq, d_model), jnp.float32),
        scratch_types=(
            [pltpu.VMEM((b_per_w,), jnp.int32)]
            + [pltpu.VMEM((chunk, d_model), jnp.float32)] * nbuf
            + [pltpu.SemaphoreType.DMA] * (2 * nbuf)
        ),
    )
    def gather_kernel(idx_hbm, table_hbm, out_hbm, idx_v, *rest):
        bufs = rest[:nbuf]
        gsems = rest[nbuf:2 * nbuf]
        ssems = rest[2 * nbuf:]

        wid = lax.axis_index("s") * _NC + lax.axis_index("c")
        row = wid // w_per_row
        col = (wid % w_per_row) * b_per_w
        pltpu.sync_copy(idx_hbm.at[row, pl.ds(col, b_per_w)], idx_v)

        def start_gather(c):
            return pltpu.async_copy(
                table_hbm.at[idx_v.at[pl.ds(c * chunk, chunk)]],
                bufs[c % nbuf],
                gsems[c % nbuf],
            )

        def start_store(c):
            return pltpu.async_copy(
                bufs[c % nbuf],
                out_hbm.at[row, pl.ds(col + c * chunk, chunk)],
                ssems[c % nbuf],
            )

        gathers = [None] * n_chunks
        stores = [None] * n_chunks
        for c in range(min(nbuf - 1, n_chunks)):
            gathers[c] = start_gather(c)
        for c in range(n_chunks):
            if c + nbuf - 1 < n_chunks:
                if c >= 1:
                    stores[c - 1].wait()
                gathers[c + nbuf - 1] = start_gather(c + nbuf - 1)
            gathers[c].wait()
            stores[c] = start_store(c)
        for c in range(max(0, n_chunks - nbuf), n_chunks):
            stores[c].wait()

    return gather_kernel


def kernel(input_ids, table):
    batch, seq = input_ids.shape
    vocab, d_model = table.shape
    return _make_gather(batch, seq, d_model)(
        input_ids.astype(jnp.int32), table)


# speedup vs baseline: 1.0254x; 1.0034x over previous
import functools

import jax
import jax.numpy as jnp
from jax import lax
from jax.experimental import pallas as pl
from jax.experimental.pallas import tpu as pltpu
from jax.experimental.pallas import tpu_sc as plsc

_INFO = plsc.get_sparse_core_info()
_NC = _INFO.num_cores
_NS = _INFO.num_subcores
_NW = _NC * _NS

_CHUNK = 16
_NBUF = 6


@functools.lru_cache(maxsize=None)
def _make_gather(batch: int, seq: int, d_model: int):
    num_rows = batch * seq
    assert num_rows % _NW == 0
    b_per_w = num_rows // _NW
    assert seq % b_per_w == 0
    w_per_row = seq // b_per_w
    chunk = _CHUNK
    nbuf = _NBUF
    assert b_per_w % chunk == 0
    n_chunks = b_per_w // chunk

    mesh = plsc.VectorSubcoreMesh(core_axis_name="c", subcore_axis_name="s")

    @functools.partial(
        pl.kernel,
        mesh=mesh,
        out_type=jax.ShapeDtypeStruct((batch, seq, d_model), jnp.float32),
        scratch_types=(
            [pltpu.VMEM((b_per_w,), jnp.int32)]
            + [pltpu.VMEM((chunk, d_model), jnp.float32)] * nbuf
            + [pltpu.SemaphoreType.DMA] * (2 * nbuf)
        ),
    )
    def gather_kernel(idx_hbm, table_hbm, out_hbm, idx_v, *rest):
        bufs = rest[:nbuf]
        gsems = rest[nbuf:2 * nbuf]
        ssems = rest[2 * nbuf:]

        wid = lax.axis_index("s") * _NC + lax.axis_index("c")
        row = wid // w_per_row
        col = (wid % w_per_row) * b_per_w
        pltpu.sync_copy(idx_hbm.at[row, pl.ds(col, b_per_w)], idx_v)

        def start_gather(c):
            return pltpu.async_copy(
                table_hbm.at[idx_v.at[pl.ds(c * chunk, chunk)]],
                bufs[c % nbuf],
                gsems[c % nbuf],
            )

        def start_store(c):
            return pltpu.async_copy(
                bufs[c % nbuf],
                out_hbm.at[row, pl.ds(col + c * chunk, chunk)],
                ssems[c % nbuf],
            )

        gathers = [None] * n_chunks
        stores = [None] * n_chunks
        for c in range(min(nbuf - 1, n_chunks)):
            gathers[c] = start_gather(c)
        for c in range(n_chunks):
            if c + nbuf - 1 < n_chunks:
                if c >= 1:
                    stores[c - 1].wait()
                gathers[c + nbuf - 1] = start_gather(c + nbuf - 1)
            gathers[c].wait()
            stores[c] = start_store(c)
        for c in range(max(0, n_chunks - nbuf), n_chunks):
            stores[c].wait()

    return gather_kernel


def kernel(input_ids, table):
    batch, seq = input_ids.shape
    vocab, d_model = table.shape
    return _make_gather(batch, seq, d_model)(
        input_ids.astype(jnp.int32), table)
